# baseline (speedup 1.0000x reference)
import functools
import math

import jax
import jax.numpy as jnp
from jax import lax
from jax.experimental import pallas as pl
from jax.experimental.pallas import tpu as pltpu
from jax.experimental.pallas import tpu_sc as plsc

_NC = 2
_NS = 16
_NW = _NC * _NS
_CH = 128


def kernel(text, token_table, pos_table):
    b, s = text.shape
    d = token_table.shape[1]
    n = b * s
    per_w = n // _NW
    nch = per_w // _CH

    period_rows = math.lcm(_CH, s)
    pchunks = period_rows // _CH
    pos_rep = jnp.tile(pos_table[:s], (period_rows // s, 1)).reshape(
        pchunks, _CH, d)

    idx = text.reshape(_NW, nch, _CH).astype(jnp.int32)

    mesh = plsc.VectorSubcoreMesh(core_axis_name="c", subcore_axis_name="s")

    @functools.partial(
        pl.kernel,
        out_type=jax.ShapeDtypeStruct((n, d), jnp.float32),
        mesh=mesh,
        scratch_types=[
            pltpu.VMEM((nch, _CH), jnp.int32),
            pltpu.VMEM((_CH, d), jnp.float32),
            pltpu.SemaphoreType.DMA,
        ],
        compiler_params=pltpu.CompilerParams(use_tc_tiling_on_sc=False),
    )
    def run(idx_hbm, tok_hbm, posrep_hbm, out_hbm, idx_v, buf, gsem):
        wid = lax.axis_index("s") * _NC + lax.axis_index("c")
        pltpu.sync_copy(idx_hbm.at[wid], idx_v)
        base = wid * per_w

        @pl.loop(0, nch)
        def chunk_loop(j):
            pltpu.sync_copy(posrep_hbm.at[lax.rem(j, pchunks)], buf)
            pltpu.async_copy(tok_hbm.at[idx_v.at[j]], buf, gsem,
                             add=True).wait()
            pltpu.sync_copy(buf, out_hbm.at[pl.ds(base + j * _CH, _CH)])

    out = run(idx, token_table, pos_rep)
    return out.reshape(b, s, d)


# speedup vs baseline: 1.1314x; 1.1314x over previous
import functools

import jax
import jax.numpy as jnp
from jax import lax
from jax.experimental import pallas as pl
from jax.experimental.pallas import tpu as pltpu
from jax.experimental.pallas import tpu_sc as plsc

_NC = 2
_NS = 16
_NW = _NC * _NS
_CH = 100
_NBUF = 4


def kernel(text, token_table, pos_table):
    b, s = text.shape
    d = token_table.shape[1]
    n = b * s
    per_w = n // _NW
    k = per_w // _CH

    idx = text.reshape(_NW, k, _CH).astype(jnp.int32)

    mesh = plsc.VectorSubcoreMesh(core_axis_name="c", subcore_axis_name="s")

    @functools.partial(
        pl.kernel,
        out_type=jax.ShapeDtypeStruct((n, d), jnp.float32),
        mesh=mesh,
        scratch_types=[
            pltpu.VMEM((k, _CH), jnp.int32),
            pltpu.VMEM_SHARED((s, d), jnp.float32),
            pltpu.VMEM((_CH, d), jnp.float32),
            pltpu.VMEM((_CH, d), jnp.float32),
            pltpu.VMEM((_CH, d), jnp.float32),
            pltpu.VMEM((_CH, d), jnp.float32),
            pltpu.SemaphoreType.DMA((_NBUF,)),
            pltpu.SemaphoreType.DMA((_NBUF,)),
            pltpu.SemaphoreType.DMA((_NBUF,)),
        ],
        compiler_params=pltpu.CompilerParams(use_tc_tiling_on_sc=False),
    )
    def run(idx_hbm, tok_hbm, pos_hbm, out_hbm, idx_v, posv,
            b0, b1, b2, b3, fsem, gsem, ssem):
        bufs = [b0, b1, b2, b3]
        sid = lax.axis_index("s")
        wid = sid * _NC + lax.axis_index("c")
        pltpu.sync_copy(idx_hbm.at[wid], idx_v)

        @pl.when(sid == 0)
        def _():
            pltpu.sync_copy(pos_hbm.at[pl.ds(0, s)], posv)

        plsc.subcore_barrier()
        base = wid * per_w

        def possrc(parity):
            return posv.at[pl.ds(parity * _CH, _CH)]

        def start_fill(parity, bi):
            pltpu.async_copy(possrc(parity), bufs[bi], fsem.at[bi])

        def wait_fill(bi):
            pltpu.make_async_copy(possrc(0), bufs[bi], fsem.at[bi]).wait()

        def start_gather(j, bi):
            pltpu.async_copy(tok_hbm.at[idx_v.at[j]], bufs[bi], gsem.at[bi],
                             add=True)

        def wait_gather(bi):
            pltpu.make_async_copy(tok_hbm.at[idx_v.at[0]], bufs[bi],
                                  gsem.at[bi]).wait()

        def start_store(j, bi):
            pltpu.async_copy(bufs[bi], out_hbm.at[pl.ds(base + j * _CH, _CH)],
                             ssem.at[bi])

        def wait_store(bi):
            pltpu.make_async_copy(bufs[bi], out_hbm.at[pl.ds(base, _CH)],
                                  ssem.at[bi]).wait()

        start_fill(0, 0)
        start_fill(1, 1)

        @pl.loop(0, k, step=_NBUF)
        def grp(g):
            for bi in range(_NBUF):
                j = g + bi
                wait_fill(bi)
                start_gather(j, bi)
                bm1 = (bi - 1) % _NBUF
                if bi >= 1:
                    wait_gather(bm1)
                    start_store(j - 1, bm1)
                else:
                    @pl.when(j >= 1)
                    def _():
                        wait_gather(bm1)
                        start_store(j - 1, bm1)
                bp2 = (bi + 2) % _NBUF
                if bi >= 2:
                    wait_store(bp2)
                else:
                    @pl.when(j >= 2)
                    def _():
                        wait_store(bp2)
                if bi < 2:
                    start_fill(bi % 2, bp2)
                else:
                    @pl.when(j + 2 < k)
                    def _():
                        start_fill(bi % 2, bp2)

        last = (k - 1) % _NBUF
        wait_gather(last)
        start_store(k - 1, last)
        wait_store((k - 2) % _NBUF)
        wait_store(last)

    out = run(idx, token_table, pos_table)
    return out.reshape(b, s, d)
